# Initial kernel scaffold; baseline (speedup 1.0000x reference)
import jax
import jax.numpy as jnp
from jax.experimental import pallas as pl


def kernel(x, lengths, table, W1, b1, W2, b2):
    raise NotImplementedError("write your pallas kernel here")


# speedup vs baseline: 6.1521x; 6.1521x over previous
import jax
import jax.numpy as jnp
from jax import lax
from jax.experimental import pallas as pl
from jax.experimental.pallas import tpu as pltpu
from jax.experimental.pallas import tpu_sc as plsc
import functools

B = 4096
SEQ = 200
D = 128
NC = 2
NS = 16
NW = NC * NS
BPW = B // NW
LPW = BPW * SEQ
CHUNK = 128
NCHUNK = LPW // CHUNK


def _make_bag_kernel():
    mesh = plsc.VectorSubcoreMesh(core_axis_name="c", subcore_axis_name="s")

    @functools.partial(
        pl.kernel,
        mesh=mesh,
        out_type=jax.ShapeDtypeStruct((B, D), jnp.float32),
        scratch_types=[
            pltpu.VMEM((NCHUNK, CHUNK), jnp.int32),
            pltpu.VMEM((NCHUNK, CHUNK), jnp.int32),
            pltpu.VMEM((CHUNK, D), jnp.float32),
            pltpu.VMEM_SHARED((NS * BPW, D), jnp.float32),
            pltpu.SemaphoreType.DMA,
        ],
    )
    def bag(x_hbm, seg_hbm, zeros_hbm, table_hbm, rep_hbm,
            idx_v, seg_v, rows_v, acc_sh, sem):
        sid = lax.axis_index("s")
        wid = sid * NC + lax.axis_index("c")
        pltpu.sync_copy(x_hbm.at[wid], idx_v)
        pltpu.sync_copy(seg_hbm.at[sid], seg_v)
        pltpu.sync_copy(zeros_hbm, acc_sh.at[pl.ds(sid * BPW, BPW)])

        def body(c, carry):
            pltpu.sync_copy(table_hbm.at[idx_v.at[c]], rows_v)
            pltpu.sync_copy(rows_v, acc_sh.at[seg_v.at[c]], add=True)
            return carry

        lax.fori_loop(0, NCHUNK, body, 0)
        pltpu.sync_copy(acc_sh.at[pl.ds(sid * BPW, BPW)],
                        rep_hbm.at[pl.ds(wid * BPW, BPW)])

    return bag


def _mlp_body(rep_ref, len_ref, w1_ref, b1_ref, w2_ref, b2_ref, out_ref):
    rep = rep_ref[...] / len_ref[...]
    h = jnp.maximum(
        jnp.dot(rep, w1_ref[...].T, preferred_element_type=jnp.float32)
        + b1_ref[...], 0.0)
    out_ref[...] = (
        jnp.dot(h, w2_ref[...].T, preferred_element_type=jnp.float32)
        + b2_ref[...])


def kernel(x, lengths, table, W1, b1, W2, b2):
    x_r = x.astype(jnp.int32).reshape(NW, NCHUNK, CHUNK)
    seg0 = (jnp.arange(LPW, dtype=jnp.int32) // SEQ).reshape(1, NCHUNK, CHUNK)
    offs = (jnp.arange(NS, dtype=jnp.int32) * BPW).reshape(NS, 1, 1)
    seg = seg0 + offs
    zeros = jnp.zeros((BPW, D), jnp.float32)

    rep = _make_bag_kernel()(x_r, seg, zeros, table)

    hidden = W1.shape[0]
    out_size = W2.shape[0]
    blk = 256
    grid = (B // blk,)
    logits = pl.pallas_call(
        _mlp_body,
        grid=grid,
        in_specs=[
            pl.BlockSpec((blk, D), lambda i: (i, 0)),
            pl.BlockSpec((blk, 1), lambda i: (i, 0)),
            pl.BlockSpec((hidden, D), lambda i: (0, 0)),
            pl.BlockSpec((1, hidden), lambda i: (0, 0)),
            pl.BlockSpec((out_size, hidden), lambda i: (0, 0)),
            pl.BlockSpec((1, out_size), lambda i: (0, 0)),
        ],
        out_specs=pl.BlockSpec((blk, out_size), lambda i: (i, 0)),
        out_shape=jax.ShapeDtypeStruct((B, out_size), jnp.float32),
    )(rep, lengths.astype(jnp.float32).reshape(B, 1),
      W1, b1.reshape(1, hidden), W2, b2.reshape(1, out_size))
    return logits


# speedup vs baseline: 9.9862x; 1.6232x over previous
import jax
import jax.numpy as jnp
from jax import lax
from jax.experimental import pallas as pl
from jax.experimental.pallas import tpu as pltpu
from jax.experimental.pallas import tpu_sc as plsc
import functools

B = 4096
SEQ = 200
D = 128
NC = 2
NS = 16
NW = NC * NS
BPW = B // NW
LPW = BPW * SEQ
CHUNK = 128
NCHUNK = LPW // CHUNK
NB = 3


def _make_bag_kernel():
    mesh = plsc.VectorSubcoreMesh(core_axis_name="c", subcore_axis_name="s")

    @functools.partial(
        pl.kernel,
        mesh=mesh,
        out_type=jax.ShapeDtypeStruct((B, D), jnp.float32),
        scratch_types=[
            pltpu.VMEM((NCHUNK, CHUNK), jnp.int32),
            pltpu.VMEM((NCHUNK, CHUNK), jnp.int32),
            pltpu.VMEM((NB, CHUNK, D), jnp.float32),
            pltpu.VMEM_SHARED((NS * BPW, D), jnp.float32),
            [pltpu.SemaphoreType.DMA] * NB,
            [pltpu.SemaphoreType.DMA] * NB,
        ],
    )
    def bag(x_hbm, seg_hbm, zeros_hbm, table_hbm, rep_hbm,
            idx_v, seg_v, rows_v, acc_sh, gsems, ssems):
        sid = lax.axis_index("s")
        wid = sid * NC + lax.axis_index("c")
        pltpu.sync_copy(x_hbm.at[wid], idx_v)
        pltpu.sync_copy(seg_hbm.at[sid], seg_v)
        pltpu.sync_copy(zeros_hbm, acc_sh.at[pl.ds(sid * BPW, BPW)])

        for b in range(NB):
            pltpu.async_copy(table_hbm.at[idx_v.at[b]], rows_v.at[b], gsems[b])

        def step(c, b):
            pltpu.make_async_copy(
                table_hbm.at[idx_v.at[c]], rows_v.at[b], gsems[b]).wait()
            pltpu.async_copy(
                rows_v.at[b], acc_sh.at[seg_v.at[c]], ssems[b], add=True)
            pltpu.make_async_copy(
                rows_v.at[b], acc_sh.at[seg_v.at[c]], ssems[b]).wait()

        NG = NCHUNK // NB

        def body(g, carry):
            for b in range(NB):
                c = g * NB + b
                step(c, b)

                @pl.when(c + NB < NCHUNK)
                def _():
                    pltpu.async_copy(
                        table_hbm.at[idx_v.at[c + NB]], rows_v.at[b], gsems[b])
            return carry

        lax.fori_loop(0, NG, body, 0)
        for b in range(NCHUNK % NB):
            step(NG * NB + b, b)
        pltpu.sync_copy(acc_sh.at[pl.ds(sid * BPW, BPW)],
                        rep_hbm.at[pl.ds(wid * BPW, BPW)])

    return bag


def _mlp_body(rep_ref, len_ref, w1_ref, b1_ref, w2_ref, b2_ref, out_ref):
    rep = rep_ref[...] / len_ref[...]
    h = jnp.maximum(
        jnp.dot(rep, w1_ref[...].T, preferred_element_type=jnp.float32)
        + b1_ref[...], 0.0)
    out_ref[...] = (
        jnp.dot(h, w2_ref[...].T, preferred_element_type=jnp.float32)
        + b2_ref[...])


def kernel(x, lengths, table, W1, b1, W2, b2):
    x_r = x.astype(jnp.int32).reshape(NW, NCHUNK, CHUNK)
    seg0 = (jnp.arange(LPW, dtype=jnp.int32) // SEQ).reshape(1, NCHUNK, CHUNK)
    offs = (jnp.arange(NS, dtype=jnp.int32) * BPW).reshape(NS, 1, 1)
    seg = seg0 + offs
    zeros = jnp.zeros((BPW, D), jnp.float32)

    rep = _make_bag_kernel()(x_r, seg, zeros, table)

    hidden = W1.shape[0]
    out_size = W2.shape[0]
    blk = 256
    grid = (B // blk,)
    logits = pl.pallas_call(
        _mlp_body,
        grid=grid,
        in_specs=[
            pl.BlockSpec((blk, D), lambda i: (i, 0)),
            pl.BlockSpec((blk, 1), lambda i: (i, 0)),
            pl.BlockSpec((hidden, D), lambda i: (0, 0)),
            pl.BlockSpec((1, hidden), lambda i: (0, 0)),
            pl.BlockSpec((out_size, hidden), lambda i: (0, 0)),
            pl.BlockSpec((1, out_size), lambda i: (0, 0)),
        ],
        out_specs=pl.BlockSpec((blk, out_size), lambda i: (i, 0)),
        out_shape=jax.ShapeDtypeStruct((B, out_size), jnp.float32),
    )(rep, lengths.astype(jnp.float32).reshape(B, 1),
      W1, b1.reshape(1, hidden), W2, b2.reshape(1, out_size))
    return logits
